# Initial kernel scaffold; baseline (speedup 1.0000x reference)
import jax
import jax.numpy as jnp
from jax.experimental import pallas as pl


def kernel(attribute_logits, attributes):
    raise NotImplementedError("write your pallas kernel here")


# speedup vs baseline: 4.6748x; 4.6748x over previous
import jax
import jax.numpy as jnp
from jax.experimental import pallas as pl
from jax.experimental.pallas import tpu as pltpu

N_ROWS = 4096
N_CLASSES = 401
MAX_ATTRS = 16
BLOCK_ROWS = 512


def _loss_block(x, ids):
    b = x.shape[0]
    lane = jax.lax.broadcasted_iota(jnp.int32, (b, N_CLASSES), 1)
    m = jnp.zeros((b, N_CLASSES), dtype=jnp.bool_)
    for j in range(MAX_ATTRS):
        aj = ids[:, j:j + 1]
        m = m | ((lane == aj) & (aj != 0))
    mf = m.astype(jnp.float32)

    mx = jnp.max(x, axis=1, keepdims=True)
    se = jnp.sum(jnp.exp(x - mx), axis=1, keepdims=True)
    lse = mx + jnp.log(se)
    g = jnp.sum(mf * x, axis=1, keepdims=True)
    d = jnp.sum(mf, axis=1, keepdims=True)
    cnt = jnp.sum((ids != 0).astype(jnp.float32), axis=1, keepdims=True)
    row_loss = jnp.where(cnt > 0, (d * lse - g) / jnp.maximum(cnt, 1.0), 0.0)
    return jnp.sum(row_loss, keepdims=True).reshape(1, 1)


def _kernel_body(logits_ref, attrs_ref, out_ref):
    @pl.when(pl.program_id(0) == 0)
    def _():
        out_ref[...] = jnp.zeros((1, 1), jnp.float32)

    s = _loss_block(logits_ref[...], attrs_ref[...])
    out_ref[...] += s * (1.0 / N_ROWS)


def kernel(attribute_logits, attributes):
    grid = N_ROWS // BLOCK_ROWS
    out = pl.pallas_call(
        _kernel_body,
        grid=(grid,),
        in_specs=[
            pl.BlockSpec((BLOCK_ROWS, N_CLASSES), lambda i: (i, 0)),
            pl.BlockSpec((BLOCK_ROWS, MAX_ATTRS), lambda i: (i, 0)),
        ],
        out_specs=pl.BlockSpec((1, 1), lambda i: (0, 0)),
        out_shape=jax.ShapeDtypeStruct((1, 1), jnp.float32),
    )(attribute_logits, attributes)
    return out[0, 0]
